# Initial kernel scaffold; baseline (speedup 1.0000x reference)
import jax
import jax.numpy as jnp
from jax.experimental import pallas as pl


def kernel(text, emb_weight, fc_w, fc_b):
    raise NotImplementedError("write your pallas kernel here")


# speedup vs baseline: 8.5535x; 8.5535x over previous
import functools

import jax
import jax.numpy as jnp
from jax import lax
from jax.experimental import pallas as pl
from jax.experimental.pallas import tpu as pltpu
from jax.experimental.pallas import tpu_sc as plsc

VOCAB = 100000
EMBED_DIM = 128
NUM_CLASS = 4
BATCH = 4096
BAG_LEN = 50

LANES = 16
NUM_WORKERS = 32
BPW = BATCH // NUM_WORKERS

ROWS_PER_BLOCK = 4000


def _proj_body(emb_ref, w_ref, out_ref):
    out_ref[...] = jnp.dot(
        emb_ref[...], w_ref[...], preferred_element_type=jnp.float32
    )


def _project(emb_weight, w_pad):
    grid = VOCAB // ROWS_PER_BLOCK
    return pl.pallas_call(
        _proj_body,
        grid=(grid,),
        in_specs=[
            pl.BlockSpec((ROWS_PER_BLOCK, EMBED_DIM), lambda i: (i, 0)),
            pl.BlockSpec((EMBED_DIM, LANES), lambda i: (0, 0)),
        ],
        out_specs=pl.BlockSpec((ROWS_PER_BLOCK, LANES), lambda i: (i, 0)),
        out_shape=jax.ShapeDtypeStruct((VOCAB, LANES), jnp.float32),
    )(emb_weight, w_pad)


_SC_MESH = plsc.VectorSubcoreMesh(
    core_axis_name="c", subcore_axis_name="s", num_cores=2, num_subcores=16
)

_CHUNK = 10


@functools.partial(
    pl.kernel,
    out_type=jax.ShapeDtypeStruct((BATCH, LANES), jnp.float32),
    mesh=_SC_MESH,
    compiler_params=pltpu.CompilerParams(use_tc_tiling_on_sc=False),
    scratch_types=[
        pltpu.VMEM((BAG_LEN, BPW), jnp.int32),
        pltpu.VMEM((LANES,), jnp.float32),
        pltpu.VMEM((BPW, LANES), jnp.float32),
        pltpu.SemaphoreType.DMA,
    ],
)
def _bag_sum(p_hbm, idx_hbm, bias_hbm, out_hbm, idx_v, bias_v, out_v, sem):
    wid = lax.axis_index("s") * 2 + lax.axis_index("c")

    pltpu.sync_copy(idx_hbm.at[wid], idx_v)
    pltpu.sync_copy(bias_hbm, bias_v)
    b16 = bias_v[...]

    @pl.loop(0, BPW)
    def _init(j):
        out_v[j] = b16

    @pl.loop(0, BAG_LEN // _CHUNK)
    def _round(ci):
        base = ci * _CHUNK
        for k in range(_CHUNK):
            pltpu.async_copy(
                p_hbm.at[idx_v.at[base + k]], out_v, sem, add=True
            )
        for _ in range(_CHUNK):
            pltpu.make_async_copy(p_hbm.at[idx_v.at[0]], out_v, sem).wait()

    pltpu.sync_copy(out_v, out_hbm.at[pl.ds(wid * BPW, BPW)])


def kernel(text, emb_weight, fc_w, fc_b):
    w_pad = (
        jnp.zeros((EMBED_DIM, LANES), jnp.float32)
        .at[:, :NUM_CLASS]
        .set(fc_w.T * (1.0 / BAG_LEN))
    )
    bias_pad = jnp.zeros((LANES,), jnp.float32).at[:NUM_CLASS].set(fc_b)
    idx = text.reshape(NUM_WORKERS, BPW, BAG_LEN).transpose(0, 2, 1)

    p = _project(emb_weight, w_pad)
    out16 = _bag_sum(p, idx, bias_pad)
    return out16[:, :NUM_CLASS]


# speedup vs baseline: 9.0181x; 1.0543x over previous
import functools

import jax
import jax.numpy as jnp
from jax import lax
from jax.experimental import pallas as pl
from jax.experimental.pallas import tpu as pltpu
from jax.experimental.pallas import tpu_sc as plsc

VOCAB = 100000
EMBED_DIM = 128
NUM_CLASS = 4
BATCH = 4096
BAG_LEN = 50

LANES = 16
NUM_WORKERS = 32
BPW = BATCH // NUM_WORKERS

ROWS_PER_BLOCK = 10000


def _proj_body(emb_ref, w_ref, out_ref):
    out_ref[...] = jnp.dot(
        emb_ref[...], w_ref[...], preferred_element_type=jnp.float32
    )


def _project(emb_weight, w_pad):
    grid = VOCAB // ROWS_PER_BLOCK
    return pl.pallas_call(
        _proj_body,
        grid=(grid,),
        in_specs=[
            pl.BlockSpec((ROWS_PER_BLOCK, EMBED_DIM), lambda i: (i, 0)),
            pl.BlockSpec((EMBED_DIM, LANES), lambda i: (0, 0)),
        ],
        out_specs=pl.BlockSpec((ROWS_PER_BLOCK, LANES), lambda i: (i, 0)),
        out_shape=jax.ShapeDtypeStruct((VOCAB, LANES), jnp.float32),
    )(emb_weight, w_pad)


_SC_MESH = plsc.VectorSubcoreMesh(
    core_axis_name="c", subcore_axis_name="s", num_cores=2, num_subcores=16
)

_CHUNK = 10


@functools.partial(
    pl.kernel,
    out_type=jax.ShapeDtypeStruct((BATCH, LANES), jnp.float32),
    mesh=_SC_MESH,
    compiler_params=pltpu.CompilerParams(use_tc_tiling_on_sc=False),
    scratch_types=[
        pltpu.VMEM((BAG_LEN, BPW), jnp.int32),
        pltpu.VMEM((LANES,), jnp.float32),
        pltpu.VMEM((BPW, LANES), jnp.float32),
        pltpu.SemaphoreType.DMA,
    ],
)
def _bag_sum(p_hbm, idx_hbm, bias_hbm, out_hbm, idx_v, bias_v, out_v, sem):
    wid = lax.axis_index("s") * 2 + lax.axis_index("c")

    pltpu.sync_copy(idx_hbm.at[wid], idx_v)
    pltpu.sync_copy(bias_hbm, bias_v)
    b16 = bias_v[...]

    @pl.loop(0, BPW)
    def _init(j):
        out_v[j] = b16

    @pl.loop(0, BAG_LEN // _CHUNK)
    def _round(ci):
        base = ci * _CHUNK
        for k in range(_CHUNK):
            pltpu.async_copy(
                p_hbm.at[idx_v.at[base + k]], out_v, sem, add=True
            )
        for _ in range(_CHUNK):
            pltpu.make_async_copy(p_hbm.at[idx_v.at[0]], out_v, sem).wait()

    pltpu.sync_copy(out_v, out_hbm.at[pl.ds(wid * BPW, BPW)])


def kernel(text, emb_weight, fc_w, fc_b):
    w_pad = (
        jnp.zeros((EMBED_DIM, LANES), jnp.float32)
        .at[:, :NUM_CLASS]
        .set(fc_w.T * (1.0 / BAG_LEN))
    )
    bias_pad = jnp.zeros((LANES,), jnp.float32).at[:NUM_CLASS].set(fc_b)
    idx = text.reshape(NUM_WORKERS, BPW, BAG_LEN).transpose(0, 2, 1)

    p = _project(emb_weight, w_pad)
    out16 = _bag_sum(p, idx, bias_pad)
    return out16[:, :NUM_CLASS]


# speedup vs baseline: 12.0969x; 1.3414x over previous
import functools

import jax
import jax.numpy as jnp
from jax import lax
from jax.experimental import pallas as pl
from jax.experimental.pallas import tpu as pltpu
from jax.experimental.pallas import tpu_sc as plsc

VOCAB = 100000
EMBED_DIM = 128
NUM_CLASS = 4
BATCH = 4096
BAG_LEN = 50

LANES = 16
NUM_WORKERS = 32
BPW = BATCH // NUM_WORKERS

PACK = EMBED_DIM // LANES
VLIN = VOCAB // PACK
LIN_BLOCK = 2504


def _proj_body(emb3_ref, w3_ref, out_ref):
    acc = jnp.dot(
        emb3_ref[:, 0, :], w3_ref[0], preferred_element_type=jnp.float32
    )
    for j in range(1, PACK):
        acc += jnp.dot(
            emb3_ref[:, j, :], w3_ref[j], preferred_element_type=jnp.float32
        )
    out_ref[...] = acc


def _project(emb3, w3):
    grid = (VLIN + LIN_BLOCK - 1) // LIN_BLOCK
    return pl.pallas_call(
        _proj_body,
        grid=(grid,),
        in_specs=[
            pl.BlockSpec((LIN_BLOCK, PACK, EMBED_DIM), lambda i: (i, 0, 0)),
            pl.BlockSpec((PACK, EMBED_DIM, EMBED_DIM), lambda i: (0, 0, 0)),
        ],
        out_specs=pl.BlockSpec((LIN_BLOCK, EMBED_DIM), lambda i: (i, 0)),
        out_shape=jax.ShapeDtypeStruct((VLIN, EMBED_DIM), jnp.float32),
    )(emb3, w3)


_SC_MESH = plsc.VectorSubcoreMesh(
    core_axis_name="c", subcore_axis_name="s", num_cores=2, num_subcores=16
)

_CHUNK = 10


@functools.partial(
    pl.kernel,
    out_type=jax.ShapeDtypeStruct((BATCH, LANES), jnp.float32),
    mesh=_SC_MESH,
    compiler_params=pltpu.CompilerParams(use_tc_tiling_on_sc=False),
    scratch_types=[
        pltpu.VMEM((BAG_LEN, BPW), jnp.int32),
        pltpu.VMEM((LANES,), jnp.float32),
        pltpu.VMEM((BPW, LANES), jnp.float32),
        pltpu.SemaphoreType.DMA,
    ],
)
def _bag_sum(p_hbm, idx_hbm, bias_hbm, out_hbm, idx_v, bias_v, out_v, sem):
    wid = lax.axis_index("s") * 2 + lax.axis_index("c")

    pltpu.sync_copy(idx_hbm.at[wid], idx_v)
    pltpu.sync_copy(bias_hbm, bias_v)
    b16 = bias_v[...]

    @pl.loop(0, BPW)
    def _init(j):
        out_v[j] = b16

    @pl.loop(0, BAG_LEN // _CHUNK)
    def _round(ci):
        base = ci * _CHUNK
        for k in range(_CHUNK):
            pltpu.async_copy(
                p_hbm.at[idx_v.at[base + k]], out_v, sem, add=True
            )
        for _ in range(_CHUNK):
            pltpu.make_async_copy(p_hbm.at[idx_v.at[0]], out_v, sem).wait()

    pltpu.sync_copy(out_v, out_hbm.at[pl.ds(wid * BPW, BPW)])


def kernel(text, emb_weight, fc_w, fc_b):
    ws = fc_w.T * (1.0 / BAG_LEN)
    w3 = jnp.zeros((PACK, EMBED_DIM, EMBED_DIM), jnp.float32)
    for j in range(PACK):
        w3 = w3.at[j, :, j * LANES : j * LANES + NUM_CLASS].set(ws)
    bias_pad = jnp.zeros((LANES,), jnp.float32).at[:NUM_CLASS].set(fc_b)
    idx = text.reshape(NUM_WORKERS, BPW, BAG_LEN).transpose(0, 2, 1)

    emb3 = emb_weight.reshape(VLIN, PACK, EMBED_DIM)
    p_lin = _project(emb3, w3)
    p = p_lin.reshape(VOCAB, LANES)
    out16 = _bag_sum(p, idx, bias_pad)
    return out16[:, :NUM_CLASS]
